# baseline (speedup 1.0000x reference)
import jax
import jax.numpy as jnp
from jax import lax
from jax.experimental import pallas as pl
from jax.experimental.pallas import tpu as pltpu
from jax.experimental.pallas import tpu_sc as plsc

L = 32
C = 2
B = 16384
NW = 32
BW = B // NW
NG = BW // 16
GI = 8
TCW = BW // 128

_LN2 = 0.6931471805599453
_BITS_TO_LN = _LN2 / (1 << 23)
_LN_OFFSET = (127.0 - 0.0430) * _LN2


def _bcast(ref, j):
    return plsc.load_gather(ref, [jnp.full((16,), j, jnp.int32)])


def _ln_seed(y):
    bits = plsc.bitcast(y, jnp.int32)
    return bits.astype(jnp.float32) * _BITS_TO_LN - _LN_OFFSET


def _ln_newton(y, x):
    return x - 1.0 + y * jnp.exp(-x)


def _eslice(ref, g, c, jb, k):
    return ref[c * 4 + jb, g >> 3, k, pl.ds((g & 7) * 16, 16)]


def _sc_body(e_hbm, coef_hbm, out_hbm, e_v, ra_v, u_v, out_v, coef_v):
    wid = lax.axis_index("s") * 2 + lax.axis_index("c")
    base = wid * TCW
    pltpu.sync_copy(e_hbm.at[:, pl.ds(base, TCW)], e_v)
    pltpu.sync_copy(coef_hbm, coef_v)

    ones = jnp.ones((16,), jnp.float32)

    def up_step(jb, k, g0, offs, rs, last):
        j = jb * 8 + k
        cs = (_bcast(coef_v, j), _bcast(coef_v, L + j),
              _bcast(coef_v, 2 * L + j)) if not last else None
        for gi in range(GI):
            g = g0 + gi
            u = jnp.exp(_eslice(e_v, g, 1, jb, k) - _eslice(e_v, g, 0, jb, k))
            u_v[pl.ds(offs[gi] + j * 16, 16)] = u
            if not last:
                c1, c2, c3 = cs
                t = u * rs[gi]
                r2 = c1 * (1.0 + c2 * t) / (1.0 + c3 * t)
                ra_v[pl.ds(offs[gi] + (j - 1) * 16, 16)] = r2
                rs[gi] = r2
        return rs

    def up_blk(gb, carry):
        g0 = gb * GI
        offs = [(g0 + gi) * (L * 16) for gi in range(GI)]
        for gi in range(GI):
            ra_v[pl.ds(offs[gi] + (L - 1) * 16, 16)] = ones

        def up_jb(i, rs_t):
            jb = 3 - i
            rs = list(rs_t)
            for k in range(7, -1, -1):
                rs = up_step(jb, k, g0, offs, rs, last=False)
            return tuple(rs)

        rs = list(plsc.parallel_loop(0, 3, carry=(ones,) * GI)(up_jb))
        for k in range(7, -1, -1):
            rs = up_step(0, k, g0, offs, rs, last=(k == 0))
        return carry

    lax.fori_loop(0, NG // GI, up_blk, 0)

    def dn_blk(gb, carry):
        g0 = gb * GI
        offs = [(g0 + gi) * (L * 16) for gi in range(GI)]

        def dn_jb(jb, rbs_t):
            rbs = list(rbs_t)
            for k in range(8):
                j = jb * 8 + k
                d1 = _bcast(coef_v, 3 * L + j)
                d2 = _bcast(coef_v, 4 * L + j)
                d3 = _bcast(coef_v, 5 * L + j)
                for gi in range(GI):
                    g = g0 + gi
                    u = u_v[pl.ds(offs[gi] + j * 16, 16)]
                    raj = ra_v[pl.ds(offs[gi] + j * 16, 16)]
                    t = u * rbs[gi]
                    q = t * raj
                    y = 1.0 + q
                    x = _ln_newton(y, _ln_seed(y))
                    xq = _ln_newton(q, _ln_seed(q))
                    out_v[0 * 4 + jb, g >> 3, k,
                          pl.ds((g & 7) * 16, 16)] = -x
                    out_v[4 + jb, g >> 3, k,
                          pl.ds((g & 7) * 16, 16)] = xq - x
                    rbs[gi] = d1 * (1.0 + d2 * t) / (1.0 + d3 * t)
            return tuple(rbs)

        plsc.parallel_loop(0, 4, unroll=2, carry=(ones,) * GI)(dn_jb)
        return carry

    lax.fori_loop(0, NG // GI, dn_blk, 0)

    pltpu.sync_copy(out_v, out_hbm.at[:, pl.ds(base, TCW)])


@jax.jit
def _sc_call(e_sc, coefs):
    mesh = plsc.VectorSubcoreMesh(core_axis_name="c", subcore_axis_name="s")
    return pl.kernel(
        _sc_body,
        mesh=mesh,
        compiler_params=pltpu.CompilerParams(needs_layout_passes=False),
        out_type=jax.ShapeDtypeStruct((C * L // 8, B // 128, 8, 128),
                                      jnp.float32),
        scratch_types=[
            pltpu.VMEM((C * L // 8, TCW, 8, 128), jnp.float32),
            pltpu.VMEM((NG * L * 16,), jnp.float32),
            pltpu.VMEM((NG * L * 16,), jnp.float32),
            pltpu.VMEM((C * L // 8, TCW, 8, 128), jnp.float32),
            pltpu.VMEM((6 * L,), jnp.float32),
        ],
    )(e_sc, coefs)


def kernel(emissions, transitions):
    i = jnp.arange(L - 1)
    t_up = transitions[i, i + 1]
    t_dn = transitions[i + 1, i]

    def mk(t):
        return (jnp.exp(t[:, 1, 0] - t[:, 0, 0]),
                jnp.exp(t[:, 1, 1] - t[:, 1, 0]),
                jnp.exp(t[:, 0, 1] - t[:, 0, 0]))

    c1, c2, c3 = mk(t_up)
    d1, d2, d3 = mk(t_dn)
    one = jnp.ones((1,), jnp.float32)
    coefs = jnp.concatenate(
        [one, c1, one, c2, one, c3, d1, one, d2, one, d3, one])

    e_sc = (emissions.transpose(1, 2, 0)
            .reshape(C, L // 8, 8, B // 128, 128)
            .transpose(0, 1, 3, 2, 4)
            .reshape(C * L // 8, B // 128, 8, 128))
    out_sc = _sc_call(e_sc, coefs)
    return (out_sc.reshape(C, L // 8, B // 128, 8, 128)
            .transpose(0, 1, 3, 2, 4)
            .reshape(C, L, B)
            .transpose(2, 0, 1))


# speedup vs baseline: 2.8764x; 2.8764x over previous
import jax
import jax.numpy as jnp
from jax import lax
from jax.experimental import pallas as pl
from jax.experimental.pallas import tpu as pltpu
from jax.experimental.pallas import tpu_sc as plsc

L = 32
C = 2
B = 16384
NW = 32
BW = B // NW
NG = BW // 16
GI = 8
TCW = BW // 128

_LN2 = 0.6931471805599453
_BITS_TO_LN = _LN2 / (1 << 23)
_LN_OFFSET = (127.0 - 0.0430) * _LN2


def _bcast(ref, j):
    return plsc.load_gather(ref, [jnp.full((16,), j, jnp.int32)])


def _ln_seed(y):
    bits = plsc.bitcast(y, jnp.int32)
    return bits.astype(jnp.float32) * _BITS_TO_LN - _LN_OFFSET


def _ln_newton(y, x):
    return x - 1.0 + y * jnp.exp(-x)


def _eslice(ref, g, c, jb, k):
    return ref[c * 4 + jb, g >> 3, k, pl.ds((g & 7) * 16, 16)]


def _sc_body(e_hbm, coef_hbm, out_hbm, e_v, ra_v, u_v, out_v, coef_v):
    wid = lax.axis_index("s") * 2 + lax.axis_index("c")
    base = wid * TCW
    pltpu.sync_copy(e_hbm.at[:, pl.ds(base, TCW)], e_v)
    pltpu.sync_copy(coef_hbm, coef_v)

    ones = jnp.ones((16,), jnp.float32)

    def up_step(jb, k, g0, offs, rs, last):
        j = jb * 8 + k
        R = range(GI)
        des = [_eslice(e_v, g0 + gi, 1, jb, k) - _eslice(e_v, g0 + gi, 0, jb, k)
               for gi in R]
        us = [jnp.exp(de) for de in des]
        for gi in R:
            u_v[pl.ds(offs[gi] + j * 16, 16)] = us[gi]
        if last:
            return rs
        c1 = _bcast(coef_v, j)
        c2 = _bcast(coef_v, L + j)
        c3 = _bcast(coef_v, 2 * L + j)
        ts = [us[gi] * rs[gi] for gi in R]
        nums = [c1 * (1.0 + c2 * t) for t in ts]
        dens = [1.0 + c3 * t for t in ts]
        r2s = [nums[gi] / dens[gi] for gi in R]
        for gi in R:
            ra_v[pl.ds(offs[gi] + (j - 1) * 16, 16)] = r2s[gi]
        return r2s

    def up_blk(gb, carry):
        g0 = gb * GI
        offs = [(g0 + gi) * (L * 16) for gi in range(GI)]
        for gi in range(GI):
            ra_v[pl.ds(offs[gi] + (L - 1) * 16, 16)] = ones

        def up_jb(i, rs_t):
            jb = 3 - i
            rs = list(rs_t)
            for k in range(7, -1, -1):
                rs = up_step(jb, k, g0, offs, rs, last=False)
            return tuple(rs)

        rs = list(plsc.parallel_loop(0, 3, carry=(ones,) * GI)(up_jb))
        for k in range(7, -1, -1):
            rs = up_step(0, k, g0, offs, rs, last=(k == 0))
        return carry

    lax.fori_loop(0, NG // GI, up_blk, 0)

    def dn_blk(gb, carry):
        g0 = gb * GI
        offs = [(g0 + gi) * (L * 16) for gi in range(GI)]

        def dn_jb(jb, rbs_t):
            rbs = list(rbs_t)
            R = range(GI)
            for k in range(8):
                j = jb * 8 + k
                d1 = _bcast(coef_v, 3 * L + j)
                d2 = _bcast(coef_v, 4 * L + j)
                d3 = _bcast(coef_v, 5 * L + j)
                us = [u_v[pl.ds(offs[gi] + j * 16, 16)] for gi in R]
                ras = [ra_v[pl.ds(offs[gi] + j * 16, 16)] for gi in R]
                ts = [us[gi] * rbs[gi] for gi in R]
                qs = [ts[gi] * ras[gi] for gi in R]
                nums = [d1 * (1.0 + d2 * t) for t in ts]
                dens = [1.0 + d3 * t for t in ts]
                rbs = [nums[gi] / dens[gi] for gi in R]
                ys = [1.0 + q for q in qs]
                x0s = [_ln_seed(y) for y in ys]
                q0s = [_ln_seed(q) for q in qs]
                exs = [jnp.exp(-x) for x in x0s]
                eqs = [jnp.exp(-x) for x in q0s]
                xs = [x0s[gi] - 1.0 + ys[gi] * exs[gi] for gi in R]
                xqs = [q0s[gi] - 1.0 + qs[gi] * eqs[gi] for gi in R]
                for gi in R:
                    g = g0 + gi
                    out_v[jb, g >> 3, k, pl.ds((g & 7) * 16, 16)] = -xs[gi]
                for gi in R:
                    g = g0 + gi
                    out_v[4 + jb, g >> 3, k,
                          pl.ds((g & 7) * 16, 16)] = xqs[gi] - xs[gi]
            return tuple(rbs)

        plsc.parallel_loop(0, 4, carry=(ones,) * GI)(dn_jb)
        return carry

    lax.fori_loop(0, NG // GI, dn_blk, 0)

    pltpu.sync_copy(out_v, out_hbm.at[:, pl.ds(base, TCW)])


@jax.jit
def _sc_call(e_sc, coefs):
    mesh = plsc.VectorSubcoreMesh(core_axis_name="c", subcore_axis_name="s")
    return pl.kernel(
        _sc_body,
        mesh=mesh,
        compiler_params=pltpu.CompilerParams(needs_layout_passes=False),
        out_type=jax.ShapeDtypeStruct((C * L // 8, B // 128, 8, 128),
                                      jnp.float32),
        scratch_types=[
            pltpu.VMEM((C * L // 8, TCW, 8, 128), jnp.float32),
            pltpu.VMEM((NG * L * 16,), jnp.float32),
            pltpu.VMEM((NG * L * 16,), jnp.float32),
            pltpu.VMEM((C * L // 8, TCW, 8, 128), jnp.float32),
            pltpu.VMEM((6 * L,), jnp.float32),
        ],
    )(e_sc, coefs)


def kernel(emissions, transitions):
    i = jnp.arange(L - 1)
    t_up = transitions[i, i + 1]
    t_dn = transitions[i + 1, i]

    def mk(t):
        return (jnp.exp(t[:, 1, 0] - t[:, 0, 0]),
                jnp.exp(t[:, 1, 1] - t[:, 1, 0]),
                jnp.exp(t[:, 0, 1] - t[:, 0, 0]))

    c1, c2, c3 = mk(t_up)
    d1, d2, d3 = mk(t_dn)
    one = jnp.ones((1,), jnp.float32)
    coefs = jnp.concatenate(
        [one, c1, one, c2, one, c3, d1, one, d2, one, d3, one])

    e_sc = (emissions.transpose(1, 2, 0)
            .reshape(C, L // 8, 8, B // 128, 128)
            .transpose(0, 1, 3, 2, 4)
            .reshape(C * L // 8, B // 128, 8, 128))
    out_sc = _sc_call(e_sc, coefs)
    return (out_sc.reshape(C, L // 8, B // 128, 8, 128)
            .transpose(0, 1, 3, 2, 4)
            .reshape(C, L, B)
            .transpose(2, 0, 1))


# speedup vs baseline: 2.8989x; 1.0078x over previous
import jax
import jax.numpy as jnp
from jax import lax
from jax.experimental import pallas as pl
from jax.experimental.pallas import tpu as pltpu
from jax.experimental.pallas import tpu_sc as plsc

L = 32
C = 2
B = 16384
NW = 32
BW = B // NW
NG = BW // 16
GI = 8
TCW = BW // 128

_LN2 = 0.6931471805599453
_BITS_TO_LN = _LN2 / (1 << 23)
_LN_OFFSET = (127.0 - 0.0430) * _LN2


def _bcast(ref, j):
    return plsc.load_gather(ref, [jnp.full((16,), j, jnp.int32)])


def _neg_ln_seed(y):
    bits = plsc.bitcast(y, jnp.int32)
    return bits.astype(jnp.float32) * (-_BITS_TO_LN) + _LN_OFFSET


def _ln_newton2(y, nx):
    return y * jnp.exp(nx) - 1.0 - nx


def _eslice(ref, g, c, jb, k):
    return ref[c * 4 + jb, g >> 3, k, pl.ds((g & 7) * 16, 16)]


def _sc_body(e_hbm, coef_hbm, out_hbm, e_v, ra_v, u_v, out_v, coef_v):
    wid = lax.axis_index("s") * 2 + lax.axis_index("c")
    base = wid * TCW
    pltpu.sync_copy(e_hbm.at[:, pl.ds(base, TCW)], e_v)
    pltpu.sync_copy(coef_hbm, coef_v)

    ones = jnp.ones((16,), jnp.float32)

    def up_step(jb, k, g0, offs, rs, last):
        j = jb * 8 + k
        R = range(GI)
        des = [_eslice(e_v, g0 + gi, 1, jb, k) - _eslice(e_v, g0 + gi, 0, jb, k)
               for gi in R]
        us = [jnp.exp(de) for de in des]
        for gi in R:
            u_v[pl.ds(offs[gi] + j * 16, 16)] = us[gi]
        if last:
            return rs
        c1 = _bcast(coef_v, j)
        c2 = _bcast(coef_v, L + j)
        c3 = _bcast(coef_v, 2 * L + j)
        ts = [us[gi] * rs[gi] for gi in R]
        nums = [c1 * (1.0 + c2 * t) for t in ts]
        dens = [1.0 + c3 * t for t in ts]
        r2s = [nums[gi] / dens[gi] for gi in R]
        for gi in R:
            ra_v[pl.ds(offs[gi] + (j - 1) * 16, 16)] = r2s[gi]
        return r2s

    def up_blk(gb, carry):
        g0 = gb * GI
        offs = [(g0 + gi) * (L * 16) for gi in range(GI)]
        for gi in range(GI):
            ra_v[pl.ds(offs[gi] + (L - 1) * 16, 16)] = ones

        def up_jb(i, rs_t):
            jb = 3 - i
            rs = list(rs_t)
            for k in range(7, -1, -1):
                rs = up_step(jb, k, g0, offs, rs, last=False)
            return tuple(rs)

        rs = list(plsc.parallel_loop(0, 3, carry=(ones,) * GI)(up_jb))
        for k in range(7, -1, -1):
            rs = up_step(0, k, g0, offs, rs, last=(k == 0))
        return carry

    lax.fori_loop(0, NG // GI, up_blk, 0)

    def dn_blk(gb, carry):
        g0 = gb * GI
        offs = [(g0 + gi) * (L * 16) for gi in range(GI)]

        def dn_jb(jb, rbs_t):
            rbs = list(rbs_t)
            R = range(GI)
            for k in range(8):
                j = jb * 8 + k
                d1 = _bcast(coef_v, 3 * L + j)
                d2 = _bcast(coef_v, 4 * L + j)
                d3 = _bcast(coef_v, 5 * L + j)
                us = [u_v[pl.ds(offs[gi] + j * 16, 16)] for gi in R]
                ras = [ra_v[pl.ds(offs[gi] + j * 16, 16)] for gi in R]
                ts = [us[gi] * rbs[gi] for gi in R]
                qs = [ts[gi] * ras[gi] for gi in R]
                nums = [d1 * (1.0 + d2 * t) for t in ts]
                dens = [1.0 + d3 * t for t in ts]
                rbs = [nums[gi] / dens[gi] for gi in R]
                ys = [1.0 + q for q in qs]
                ny = [_neg_ln_seed(y) for y in ys]
                nq = [_neg_ln_seed(q) for q in qs]
                xs = [_ln_newton2(ys[gi], ny[gi]) for gi in R]
                xqs = [_ln_newton2(qs[gi], nq[gi]) for gi in R]
                for gi in R:
                    g = g0 + gi
                    out_v[jb, g >> 3, k, pl.ds((g & 7) * 16, 16)] = -xs[gi]
                for gi in R:
                    g = g0 + gi
                    out_v[4 + jb, g >> 3, k,
                          pl.ds((g & 7) * 16, 16)] = xqs[gi] - xs[gi]
            return tuple(rbs)

        plsc.parallel_loop(0, 4, carry=(ones,) * GI)(dn_jb)
        return carry

    lax.fori_loop(0, NG // GI, dn_blk, 0)

    pltpu.sync_copy(out_v, out_hbm.at[:, pl.ds(base, TCW)])


@jax.jit
def _sc_call(e_sc, coefs):
    mesh = plsc.VectorSubcoreMesh(core_axis_name="c", subcore_axis_name="s")
    return pl.kernel(
        _sc_body,
        mesh=mesh,
        compiler_params=pltpu.CompilerParams(needs_layout_passes=False),
        out_type=jax.ShapeDtypeStruct((C * L // 8, B // 128, 8, 128),
                                      jnp.float32),
        scratch_types=[
            pltpu.VMEM((C * L // 8, TCW, 8, 128), jnp.float32),
            pltpu.VMEM((NG * L * 16,), jnp.float32),
            pltpu.VMEM((NG * L * 16,), jnp.float32),
            pltpu.VMEM((C * L // 8, TCW, 8, 128), jnp.float32),
            pltpu.VMEM((6 * L,), jnp.float32),
        ],
    )(e_sc, coefs)


def kernel(emissions, transitions):
    i = jnp.arange(L - 1)
    t_up = transitions[i, i + 1]
    t_dn = transitions[i + 1, i]

    def mk(t):
        return (jnp.exp(t[:, 1, 0] - t[:, 0, 0]),
                jnp.exp(t[:, 1, 1] - t[:, 1, 0]),
                jnp.exp(t[:, 0, 1] - t[:, 0, 0]))

    c1, c2, c3 = mk(t_up)
    d1, d2, d3 = mk(t_dn)
    one = jnp.ones((1,), jnp.float32)
    coefs = jnp.concatenate(
        [one, c1, one, c2, one, c3, d1, one, d2, one, d3, one])

    e_sc = (emissions.transpose(1, 2, 0)
            .reshape(C, L // 8, 8, B // 128, 128)
            .transpose(0, 1, 3, 2, 4)
            .reshape(C * L // 8, B // 128, 8, 128))
    out_sc = _sc_call(e_sc, coefs)
    return (out_sc.reshape(C, L // 8, B // 128, 8, 128)
            .transpose(0, 1, 3, 2, 4)
            .reshape(C, L, B)
            .transpose(2, 0, 1))
